# Initial kernel scaffold; baseline (speedup 1.0000x reference)
import jax
import jax.numpy as jnp
from jax.experimental import pallas as pl


def kernel(states, x_emb, y_emb, d_emb):
    raise NotImplementedError("write your pallas kernel here")


# speedup vs baseline: 4.5228x; 4.5228x over previous
import jax
import jax.numpy as jnp
from jax.experimental import pallas as pl

_B = 2048


def _body(states_ref, x_ref, y_ref, d_ref, out_ref):
    s = states_ref[...]
    s0 = s[:, 0:1]
    s1 = s[:, 1:2]
    s2 = s[:, 2:3]
    xe = x_ref[...]
    ye = y_ref[...]
    de = d_ref[...]

    def lookup(col, table, rows):
        acc = (col == 0).astype(jnp.float32) * table[0:1, :]
        for k in range(1, rows):
            acc += (col == k).astype(jnp.float32) * table[k:k + 1, :]
        return acc

    ox = lookup(s0, xe, 10)
    oy = lookup(s1, ye, 10)
    od = lookup(s2, de, 4)
    out_ref[...] = jnp.concatenate([ox, oy, od], axis=-1)


def kernel(states, x_emb, y_emb, d_emb):
    n = states.shape[0]
    return pl.pallas_call(
        _body,
        grid=(n // _B,),
        in_specs=[
            pl.BlockSpec((_B, 3), lambda i: (i, 0)),
            pl.BlockSpec((10, 12), lambda i: (0, 0)),
            pl.BlockSpec((10, 12), lambda i: (0, 0)),
            pl.BlockSpec((4, 6), lambda i: (0, 0)),
        ],
        out_specs=pl.BlockSpec((_B, 30), lambda i: (i, 0)),
        out_shape=jax.ShapeDtypeStruct((n, 30), jnp.float32),
    )(states, x_emb, y_emb, d_emb)
